# baseline (speedup 1.0000x reference)
import jax
import jax.numpy as jnp
from jax import lax
from jax.experimental import pallas as pl
from jax.experimental.pallas import tpu as pltpu

_B, _C, _H, _W = 32, 64, 32, 32
_K = 1024
_HW = _H * _W
_RB = 1024
_NROWS = _B * _HW


def _vq_block_kernel(lat_ref, cb_ref, q_ref, idx_ref, loss_ref):
    f = lat_ref[0].reshape(_C, _RB)
    cbt = cb_ref[...].T
    cbt2 = -2.0 * cbt
    sq = (f * f).reshape(8, 8, _RB)
    t = ((sq[:, 0] + sq[:, 1]) + (sq[:, 2] + sq[:, 3])) + (
        (sq[:, 4] + sq[:, 5]) + (sq[:, 6] + sq[:, 7]))
    fsq = t[0]
    for g in range(1, 8):
        fsq = fsq + t[g]
    fsq = fsq[None, :]
    csq = jnp.sum(cbt * cbt, axis=0)[:, None]
    prod2 = lax.dot_general(cbt2, f, (((0,), (0,)), ((), ())),
                            preferred_element_type=jnp.float32)
    dist = jnp.sqrt(jnp.maximum((fsq + prod2) + csq, 0.0))
    m = jnp.min(dist, axis=0, keepdims=True)
    iota = lax.broadcasted_iota(jnp.int32, (_K, _RB), 0)
    idx = jnp.min(jnp.where(dist == m, iota, _K), axis=0)
    onehot = (iota == idx[None, :]).astype(jnp.float32)
    qt = jnp.dot(cbt, onehot, preferred_element_type=jnp.float32)
    q_ref[0] = (f + (qt - f)).reshape(_C, _H, _W)
    idx_ref[0, 0] = idx
    diff = qt - f
    loss_ref[0, 0] = jnp.full((_RB,), jnp.sum(diff * diff), jnp.float32)


def kernel(latents, codebook):
    grid = (_B,)
    q, idx, loss_p = pl.pallas_call(
        _vq_block_kernel,
        grid=grid,
        in_specs=[
            pl.BlockSpec((1, _C, _H, _W), lambda i: (i, 0, 0, 0)),
            pl.BlockSpec((_K, _C), lambda i: (0, 0)),
        ],
        out_specs=[
            pl.BlockSpec((1, _C, _H, _W), lambda i: (i, 0, 0, 0)),
            pl.BlockSpec((1, 1, _RB), lambda i: (i, 0, 0)),
            pl.BlockSpec((1, 1, _RB), lambda i: (i, 0, 0)),
        ],
        out_shape=[
            jax.ShapeDtypeStruct((_B, _C, _H, _W), jnp.float32),
            jax.ShapeDtypeStruct((_B, 1, _RB), jnp.int32),
            jax.ShapeDtypeStruct((_B, 1, _RB), jnp.float32),
        ],
        compiler_params=pltpu.CompilerParams(
            dimension_semantics=("parallel",)),
    )(latents, codebook)
    vq_loss = loss_p[:, 0, 0].sum() * (1.25 / _NROWS / _C)
    return q, vq_loss, idx.reshape(_NROWS)


# speedup vs baseline: 1.1170x; 1.1170x over previous
import jax
import jax.numpy as jnp
from jax import lax
from jax.experimental import pallas as pl
from jax.experimental.pallas import tpu as pltpu

_B, _C, _H, _W = 32, 64, 32, 32
_K = 1024
_HW = _H * _W
_RB = 1024
_NROWS = _B * _HW


def _vq_block_kernel(lat_ref, cbt2_ref, cbt_ref, q_ref, idx_ref, loss_ref):
    f = lat_ref[0].reshape(_C, _RB)
    cbt2 = cbt2_ref[...]
    cbt = cbt_ref[...]
    sq = (f * f).reshape(8, 8, _RB)
    t = ((sq[:, 0] + sq[:, 1]) + (sq[:, 2] + sq[:, 3])) + (
        (sq[:, 4] + sq[:, 5]) + (sq[:, 6] + sq[:, 7]))
    fsq = t[0]
    for g in range(1, 8):
        fsq = fsq + t[g]
    fsq = fsq[None, :]
    csq = jnp.sum(cbt * cbt, axis=0)[:, None]
    prod2 = lax.dot_general(cbt2, f, (((0,), (0,)), ((), ())),
                            preferred_element_type=jnp.float32)
    dist = jnp.sqrt(jnp.maximum((fsq + prod2) + csq, 0.0))
    idx = jnp.argmin(dist, axis=0).astype(jnp.int32)
    iota = lax.broadcasted_iota(jnp.int32, (_K, _RB), 0)
    onehot = (iota == idx[None, :]).astype(jnp.float32)
    qt = jnp.dot(cbt, onehot, preferred_element_type=jnp.float32)
    q_ref[0] = (f + (qt - f)).reshape(_C, _H, _W)
    idx_ref[0, 0] = idx
    diff = qt - f
    loss_ref[0, 0] = jnp.full((_RB,), jnp.sum(diff * diff), jnp.float32)


def kernel(latents, codebook):
    cbt = codebook.T
    cbt2 = -2.0 * cbt
    grid = (_B,)
    q, idx, loss_p = pl.pallas_call(
        _vq_block_kernel,
        grid=grid,
        in_specs=[
            pl.BlockSpec((1, _C, _H, _W), lambda i: (i, 0, 0, 0)),
            pl.BlockSpec((_C, _K), lambda i: (0, 0)),
            pl.BlockSpec((_C, _K), lambda i: (0, 0)),
        ],
        out_specs=[
            pl.BlockSpec((1, _C, _H, _W), lambda i: (i, 0, 0, 0)),
            pl.BlockSpec((1, 1, _RB), lambda i: (i, 0, 0)),
            pl.BlockSpec((1, 1, _RB), lambda i: (i, 0, 0)),
        ],
        out_shape=[
            jax.ShapeDtypeStruct((_B, _C, _H, _W), jnp.float32),
            jax.ShapeDtypeStruct((_B, 1, _RB), jnp.int32),
            jax.ShapeDtypeStruct((_B, 1, _RB), jnp.float32),
        ],
        compiler_params=pltpu.CompilerParams(
            dimension_semantics=("parallel",)),
    )(latents, cbt2, cbt)
    vq_loss = loss_p[:, 0, 0].sum() * (1.25 / _NROWS / _C)
    return q, vq_loss, idx.reshape(_NROWS)
